# baseline (speedup 1.0000x reference)
import functools

import jax
import jax.numpy as jnp
from jax import lax
from jax.experimental import pallas as pl
from jax.experimental.pallas import tpu as pltpu
from jax.experimental.pallas import tpu_sc as plsc

N = 10000
E = 320000
D = 128

NC = 2
NS = 16
NW = NC * NS
CH = 80
NCHUNK = E // CH
CPT = NCHUNK // NW
DCH = 128

ZCH = 5
ZSTRIDE = 640

_INV_BN = (1.0 + 1e-5) ** -0.5

_mesh = plsc.VectorSubcoreMesh(core_axis_name="c", subcore_axis_name="s")


@functools.partial(
    pl.kernel,
    out_type=jax.ShapeDtypeStruct((NC * N,), jnp.float32),
    mesh=_mesh,
    scratch_types=[
        pltpu.VMEM_SHARED((N,), jnp.float32),
        pltpu.VMEM((CPT * CH,), jnp.int32),
        pltpu.VMEM((DCH,), jnp.float32),
        pltpu.VMEM((DCH,), jnp.float32),
        pltpu.VMEM((DCH,), jnp.float32),
        pltpu.SemaphoreType.DMA,
    ],
)
def _deg_sc(e_hbm, out_hbm, acc, didx, ones_v, zero_v, wb_v, ss):
    c = lax.axis_index("c")
    s = lax.axis_index("s")
    wid = s * NC + c

    for k in range(DCH // 16):
        ones_v[pl.ds(k * 16, 16)] = jnp.ones((16,), jnp.float32)
        zero_v[pl.ds(k * 16, 16)] = jnp.zeros((16,), jnp.float32)

    for k in range(ZCH):
        off = jnp.minimum(s * ZSTRIDE + k * DCH, N - DCH)
        pltpu.sync_copy(zero_v, acc.at[pl.ds(off, DCH)])
    plsc.subcore_barrier()

    pltpu.sync_copy(e_hbm.at[pl.ds(E + wid * CPT * CH, CPT * CH)], didx)

    K = 5
    def body(t, carry):
        for b in range(K):
            pltpu.async_copy(
                ones_v.at[pl.ds(0, CH)],
                acc.at[didx.at[pl.ds((t * K + b) * CH, CH)]], ss, add=True)
        for b in range(K):
            pltpu.make_async_copy(
                ones_v.at[pl.ds(0, CH)],
                acc.at[didx.at[pl.ds((t * K + b) * CH, CH)]], ss).wait()
        return carry

    lax.fori_loop(0, CPT // K, body, 0)

    plsc.subcore_barrier()
    for k in range(ZCH):
        off = jnp.minimum(s * ZSTRIDE + k * DCH, N - DCH)
        pltpu.sync_copy(acc.at[pl.ds(off, DCH)], wb_v)
        pltpu.sync_copy(wb_v, out_hbm.at[pl.ds(c * N + off, DCH)])


_UNROLL = 6


@functools.partial(
    pl.kernel,
    out_type=jax.ShapeDtypeStruct((NC, N, D), jnp.float32),
    mesh=_mesh,
    scratch_types=(
        [pltpu.VMEM_SHARED((N, D), jnp.float32)]
        + [pltpu.VMEM((CPT * CH,), jnp.int32)]
        + [pltpu.VMEM((CH,), jnp.int32)] * 6
        + [pltpu.VMEM((CH, D), jnp.float32)] * 3
        + [pltpu.SemaphoreType.DMA] * 12
    ),
)
def _mp_sc(h_hbm, e_hbm, out_hbm, acc, sidx, *bufs):
    didx = bufs[0:6]
    rows = bufs[6:9]
    sd = bufs[9:15]
    sg = bufs[15:18]
    ss = bufs[18:21]
    c = lax.axis_index("c")
    s = lax.axis_index("s")
    wid = s * NC + c
    sbase = wid * CPT * CH
    dbase = E + sbase

    def zbody(r, carry):
        for k in range(D // 16):
            rows[0][r, pl.ds(k * 16, 16)] = jnp.zeros((16,), jnp.float32)
        return carry

    lax.fori_loop(0, CH, zbody, 0)
    for k in range(ZCH):
        off = jnp.minimum(s * ZSTRIDE + k * DCH, N - DCH)
        pltpu.sync_copy(rows[0], acc.at[pl.ds(off, CH)])
        pltpu.sync_copy(rows[0], acc.at[pl.ds(off + (DCH - CH), CH)])
    plsc.subcore_barrier()

    pltpu.sync_copy(e_hbm.at[pl.ds(sbase, CPT * CH)], sidx)

    def pf_didx(j, u):
        pltpu.async_copy(e_hbm.at[pl.ds(dbase + j * CH, CH)], didx[u],
                         sd[u])

    def wait_didx(j, u):
        pltpu.make_async_copy(e_hbm.at[pl.ds(dbase + j * CH, CH)],
                              didx[u], sd[u]).wait()

    def gather(j, r):
        pltpu.async_copy(h_hbm.at[sidx.at[pl.ds(j * CH, CH)]], rows[r],
                         sg[r])

    def wait_gather(j, r):
        pltpu.make_async_copy(h_hbm.at[sidx.at[pl.ds(j * CH, CH)]],
                              rows[r], sg[r]).wait()

    def scatter(u, r):
        pltpu.async_copy(rows[r], acc.at[didx[u]], ss[r], add=True)

    def wait_scatter(u, r):
        pltpu.make_async_copy(rows[r], acc.at[didx[u]], ss[r]).wait()

    for i in range(4):
        pf_didx(i, i)
    gather(0, 0)

    NMAIN = (CPT - 5) // _UNROLL

    def body(tt, carry):
        for u in range(_UNROLL):
            j = _UNROLL * tt + u
            r = u % 3
            r2 = (u + 1) % 3
            wait_gather(j, r)
            wait_didx(j, u)
            scatter(u, r)
            if u >= 2:
                wait_scatter((u - 2) % 6, r2)
            else:
                @pl.when(tt > 0)
                def _():
                    wait_scatter((u + 4) % 6, r2)
            pf_didx(j + 4, (u + 4) % 6)
            gather(j + 1, r2)
        return carry

    lax.fori_loop(0, NMAIN, body, 0)

    for j in range(NMAIN * _UNROLL, CPT):
        u = j % 6
        r = j % 3
        r2 = (j + 1) % 3
        wait_gather(j, r)
        wait_didx(j, u)
        scatter(u, r)
        wait_scatter((j - 2) % 6, r2)
        if j + 4 < CPT:
            pf_didx(j + 4, (j + 4) % 6)
        if j + 1 < CPT:
            gather(j + 1, r2)
    wait_scatter((CPT - 2) % 6, (CPT - 2) % 3)
    wait_scatter((CPT - 1) % 6, (CPT - 1) % 3)

    plsc.subcore_barrier()
    for k in range(ZCH):
        off = jnp.minimum(s * ZSTRIDE + k * DCH, N - DCH)
        pltpu.sync_copy(acc.at[pl.ds(off, DCH)],
                        out_hbm.at[c, pl.ds(off, DCH)])


R = 2000


def _dinv_of(cnt_ref):
    seg = cnt_ref[...]
    return lax.rsqrt(1.0 + seg[:, 0] + seg[:, 1])[:, None]


def _tc1_body(cnt_ref, x_ref, w1_ref, h1s_ref):
    h = jnp.dot(x_ref[...], w1_ref[...], preferred_element_type=jnp.float32)
    h1s_ref[...] = h * _dinv_of(cnt_ref)


def _tc2_body(cnt_ref, acc_ref, h1s_ref, b1_ref, g_ref, be_ref, w2_ref,
              h2s_ref):
    dinv = _dinv_of(cnt_ref)
    s1 = acc_ref[0] + acc_ref[1] + h1s_ref[...]
    gcn1 = s1 * dinv + b1_ref[...]
    hh = jnp.maximum(gcn1 * (g_ref[...] * _INV_BN) + be_ref[...], 0.0)
    h2s_ref[...] = jnp.dot(hh, w2_ref[...],
                           preferred_element_type=jnp.float32) * dinv


def _tc3_body(cnt_ref, acc_ref, h2s_ref, b2_ref, x_ref, out_ref):
    dinv = _dinv_of(cnt_ref)
    s2 = acc_ref[0] + acc_ref[1] + h2s_ref[...]
    out_ref[...] = s2 * dinv + b2_ref[...] + x_ref[...]


_cnt_spec = pl.BlockSpec((R, 2), lambda i: (i, 0))
_row_spec = pl.BlockSpec((R, D), lambda i: (i, 0))
_acc_spec = pl.BlockSpec((2, R, D), lambda i: (0, i, 0))
_w_spec = pl.BlockSpec((D, D), lambda i: (0, 0))
_vec_spec = pl.BlockSpec((1, D), lambda i: (0, 0))

_tc1 = pl.pallas_call(
    _tc1_body, grid=(N // R,),
    in_specs=[_cnt_spec, _row_spec, _w_spec],
    out_specs=_row_spec,
    out_shape=jax.ShapeDtypeStruct((N, D), jnp.float32),
)

_tc2 = pl.pallas_call(
    _tc2_body, grid=(N // R,),
    in_specs=[_cnt_spec, _acc_spec, _row_spec, _vec_spec, _vec_spec,
              _vec_spec, _w_spec],
    out_specs=_row_spec,
    out_shape=jax.ShapeDtypeStruct((N, D), jnp.float32),
)

_tc3 = pl.pallas_call(
    _tc3_body, grid=(N // R,),
    in_specs=[_cnt_spec, _acc_spec, _row_spec, _vec_spec, _row_spec],
    out_specs=_row_spec,
    out_shape=jax.ShapeDtypeStruct((N, D), jnp.float32),
)


def kernel(x, edge_index, W1, b1, gamma, beta, W2, b2):
    eflat = edge_index.reshape(2 * E)
    cnt = _deg_sc(eflat).reshape(NC, N).T
    h1s = _tc1(cnt, x, W1)
    acc1 = _mp_sc(h1s, eflat)
    h2s = _tc2(cnt, acc1, h1s, b1.reshape(1, D), gamma.reshape(1, D),
               beta.reshape(1, D), W2)
    acc2 = _mp_sc(h2s, eflat)
    out = _tc3(cnt, acc2, h2s, b2.reshape(1, D), x)
    return out
